# baseline (speedup 1.0000x reference)
import jax
import jax.numpy as jnp
from jax.experimental import pallas as pl

_K = 5


def _cheb_kernel(a_ref, x_ref, w1_ref, b1_ref, w2_ref, b2_ref, out_ref):
    A = a_ref[...]
    N = A.shape[0]
    nb = x_ref.shape[0]
    C = w1_ref.shape[1]

    d = jnp.sum(A, axis=1)
    inv = jnp.where(d > 0, 1.0 / jnp.sqrt(d), 0.0)
    row = jax.lax.broadcasted_iota(jnp.int32, (N, N), 0)
    col = jax.lax.broadcasted_iota(jnp.int32, (N, N), 1)
    eye = jnp.where(row == col, jnp.float32(1.0), jnp.float32(0.0))
    L = eye - inv[:, None] * A * inv[None, :]

    def layer(X, w_ref, b_ref):
        def proj(T, k):
            w = w_ref[k]
            cols = [jnp.dot(T[:, b * C:(b + 1) * C], w,
                            preferred_element_type=jnp.float32)
                    for b in range(nb)]
            return jnp.concatenate(cols, axis=1)

        acc = proj(X, 0)
        T0 = X
        T1 = jnp.dot(L, X, preferred_element_type=jnp.float32)
        acc = acc + proj(T1, 1)
        for k in range(2, _K):
            T2 = 2.0 * jnp.dot(L, T1, preferred_element_type=jnp.float32) - T0
            acc = acc + proj(T2, k)
            T0, T1 = T1, T2
        return jnp.maximum(acc + b_ref[...], 0.0)

    X = jnp.concatenate([x_ref[b] for b in range(nb)], axis=1)
    h = layer(X, w1_ref, b1_ref)
    out = layer(h, w2_ref, b2_ref)
    for b in range(nb):
        out_ref[b] = out[:, b * C:(b + 1) * C]


def kernel(graph, flow_x, W1, b1, W2, b2):
    B, N, H, D = flow_x.shape
    C = H * D
    x = flow_x.reshape(B, N, C)
    out = pl.pallas_call(
        _cheb_kernel,
        out_shape=jax.ShapeDtypeStruct((B, N, C), jnp.float32),
    )(graph, x, W1, jnp.tile(b1, B).reshape(1, -1), W2,
      jnp.tile(b2, B).reshape(1, -1))
    return out.reshape(B, N, 1, C)


# speedup vs baseline: 1.2844x; 1.2844x over previous
import jax
import jax.numpy as jnp
from jax.experimental import pallas as pl

_K = 5


def _cheb_kernel(a_ref, x_ref, w1_ref, b1_ref, w2_ref, b2_ref, out_ref):
    A = a_ref[...]
    N = A.shape[0]
    BC = x_ref.shape[1]
    P = w1_ref.shape[1]
    npair = BC // P

    d = jnp.sum(A, axis=1)
    inv = jnp.where(d > 0, 1.0 / jnp.sqrt(d), 0.0)
    row = jax.lax.broadcasted_iota(jnp.int32, (N, N), 0)
    col = jax.lax.broadcasted_iota(jnp.int32, (N, N), 1)
    eye = jnp.where(row == col, jnp.float32(1.0), jnp.float32(0.0))
    L = eye - inv[:, None] * A * inv[None, :]

    def layer(X, w_ref, b_ref):
        def proj(T, k):
            w = w_ref[k]
            cols = [jnp.dot(T[:, p * P:(p + 1) * P], w,
                            preferred_element_type=jnp.float32)
                    for p in range(npair)]
            return jnp.concatenate(cols, axis=1)

        acc = proj(X, 0)
        T0 = X
        T1 = jnp.dot(L, X, preferred_element_type=jnp.float32)
        acc = acc + proj(T1, 1)
        for k in range(2, _K):
            T2 = 2.0 * jnp.dot(L, T1, preferred_element_type=jnp.float32) - T0
            acc = acc + proj(T2, k)
            T0, T1 = T1, T2
        return jnp.maximum(acc + b_ref[...], 0.0)

    h = layer(x_ref[...], w1_ref, b1_ref)
    out_ref[...] = layer(h, w2_ref, b2_ref)


def _pairblock(W):
    K, C, _ = W.shape
    z = jnp.zeros((K, C, C), W.dtype)
    top = jnp.concatenate([W, z], axis=2)
    bot = jnp.concatenate([z, W], axis=2)
    return jnp.concatenate([top, bot], axis=1)


def kernel(graph, flow_x, W1, b1, W2, b2):
    B, N, H, D = flow_x.shape
    C = H * D
    x = flow_x.reshape(B, N, C).transpose(1, 0, 2).reshape(N, B * C)
    out = pl.pallas_call(
        _cheb_kernel,
        out_shape=jax.ShapeDtypeStruct((N, B * C), jnp.float32),
    )(graph, x, _pairblock(W1), jnp.tile(b1, B).reshape(1, -1),
      _pairblock(W2), jnp.tile(b2, B).reshape(1, -1))
    return out.reshape(N, B, C).transpose(1, 0, 2)[:, :, None, :]
